# baseline (speedup 1.0000x reference)
import dataclasses
import jax
import jax.numpy as jnp
from jax import lax
from jax.experimental import pallas as pl
from jax.experimental.pallas import tpu as pltpu
from jax.experimental.pallas import tpu_sc as plsc

N_NODES_ = 10000
D_ = 128
N_EDGES_ = 320000
ROW_BLK = 2000

NW = 32
BN = 320
EPW = N_EDGES_ // NW
CAP = EPW
BIG = NW * NW * CAP + 128
DUMP = NW * NW * CAP
ACC_ROWS = BN + 1
AGG_PAD = NW * BN

_mesh = plsc.VectorSubcoreMesh(core_axis_name="c", subcore_axis_name="s")
_cp = pltpu.CompilerParams()
if "needs_layout_passes" in pltpu.CompilerParams.__dataclass_fields__:
    _cp = dataclasses.replace(_cp, needs_layout_passes=False)


def _partition(src, dst):
    @pl.kernel(
        out_type=(
            jax.ShapeDtypeStruct((BIG,), jnp.int32),
            jax.ShapeDtypeStruct((NW * NW,), jnp.int32),
        ),
        mesh=_mesh,
        compiler_params=_cp,
        scratch_types=[
            pltpu.VMEM((2000,), jnp.int32),
            pltpu.VMEM((2000,), jnp.int32),
            pltpu.VMEM((16, 128), jnp.int32),
            pltpu.VMEM((16, 128), jnp.int32),
            pltpu.VMEM((32,), jnp.int32),
            pltpu.VMEM((1, 32), jnp.int32),
            pltpu.SemaphoreType.DMA,
        ],
    )
    def k(src_hbm, dst_hbm, bkt_hbm, cnt_hbm, sv, dv, posb, pakb, ctr, cposb, sem):
        wid = lax.axis_index("s") * 2 + lax.axis_index("c")
        iota = lax.iota(jnp.int32, 16)
        zeros16 = jnp.zeros((16,), jnp.int32)
        ctr[pl.ds(0, 16)] = zeros16
        ctr[pl.ds(16, 16)] = zeros16
        base_e = wid * EPW
        for w in range(5):
            pltpu.sync_copy(src_hbm.at[pl.ds(base_e + w * 2000, 2000)], sv)
            pltpu.sync_copy(dst_hbm.at[pl.ds(base_e + w * 2000, 2000)], dv)
            for c in (80, 96, 112):
                posb[15, pl.ds(c, 16)] = zeros16 + DUMP

            def body(v, _):
                s16 = sv[pl.ds(16 * v, 16)]
                d16 = dv[pl.ds(16 * v, 16)]
                b16 = (d16 * 6554) >> 21
                dl16 = d16 - b16 * BN
                cnt16, last16 = plsc.scan_count(b16)
                prev16 = plsc.load_gather(ctr, [b16])
                pos16 = (wid * 32 + b16) * CAP + prev16 + cnt16 - 1
                plsc.addupdate_scatter(ctr, [b16], cnt16, mask=last16)
                pk16 = (s16 << 9) | dl16
                r = v >> 3
                col = (v & 7) * 16
                posb[r, pl.ds(col, 16)] = pos16
                pakb[r, pl.ds(col, 16)] = pk16
                return 0

            lax.fori_loop(0, 125, body, 0)
            cps = [
                pltpu.async_copy(pakb.at[j], bkt_hbm.at[posb.at[j]], sem)
                for j in range(16)
            ]
            for cp in cps:
                cp.wait()
        cposb[0, pl.ds(0, 16)] = iota * 32 + wid
        cposb[0, pl.ds(16, 16)] = (iota + 16) * 32 + wid
        pltpu.async_copy(ctr, cnt_hbm.at[cposb.at[0]], sem).wait()

    return k(src, dst)


def _aggregate(feats, bkt, cnt):
    @pl.kernel(
        out_type=jax.ShapeDtypeStruct((AGG_PAD * D_,), jnp.float32),
        mesh=_mesh,
        compiler_params=_cp,
        scratch_types=[
            pltpu.VMEM((ACC_ROWS * D_,), jnp.float32),
            pltpu.VMEM((128, D_), jnp.float32),
            pltpu.VMEM((128,), jnp.int32),
            pltpu.VMEM((1, 128), jnp.int32),
            pltpu.VMEM((128,), jnp.int32),
            pltpu.VMEM((1056,), jnp.int32),
            pltpu.SemaphoreType.DMA,
        ],
    )
    def k(f_hbm, bkt_hbm, cnt_hbm, agg_hbm, acc, rows, pkv, idxb, dlv, cntv, sem):
        b = lax.axis_index("s") * 2 + lax.axis_index("c")
        iota = lax.iota(jnp.int32, 16)
        pltpu.sync_copy(cnt_hbm, cntv.at[pl.ds(0, NW * NW)])
        neg = jnp.full((16,), -jnp.inf, jnp.float32)

        def initbody(i, _):
            acc[pl.ds(16 * i, 16)] = neg
            return 0

        lax.fori_loop(0, ACC_ROWS * D_ // 16, initbody, 0)

        def wbody(w, _):
            n = cntv[pl.ds(b * 32 + w, 16)][0]
            region = (w * 32 + b) * CAP
            nch = (n + 127) >> 7

            def cbody(i, _):
                cn = n - i * 128
                for vv in range(8):
                    pk = pkv[pl.ds(16 * vv, 16)]
                    act = (iota + 16 * vv) < cn
                    idxb[0, pl.ds(16 * vv, 16)] = jnp.where(act, pk * 0, 0)
                    dlv[pl.ds(16 * vv, 16)] = jnp.where(act, pk & 511, BN)
                pltpu.async_copy(f_hbm.at[idxb.at[0]], rows, sem).wait()

                def rbody(vb, _):
                    dl16 = dlv[pl.ds(16 * vb, 16)]
                    if True:
                        return 0
                    for lane in range(16):
                        d = dl16[lane]
                        ab = d * D_
                        e = 16 * vb + lane
                        for c8 in range(8):
                            a = acc[pl.ds(ab + 16 * c8, 16)]
                            r = rows[e, pl.ds(16 * c8, 16)]
                            acc[pl.ds(ab + 16 * c8, 16)] = jnp.maximum(a, r)
                    return 0

                lax.fori_loop(0, 8, rbody, 0)
                return 0

            lax.fori_loop(0, nch, cbody, 0)
            return 0

        lax.fori_loop(0, NW, wbody, 0)
        pltpu.sync_copy(
            acc.at[pl.ds(0, BN * D_)], agg_hbm.at[pl.ds(b * BN * D_, BN * D_)]
        )

    return k(feats, bkt, cnt)


def _layer1_kernel(x_ref, agg_ref, Wl_ref, bl_ref, Wr_ref, o_ref):
    agg = agg_ref[...]
    agg = jnp.where(jnp.isfinite(agg), agg, 0.0)
    o = (
        jnp.dot(agg, Wl_ref[...].T, preferred_element_type=jnp.float32)
        + bl_ref[...]
        + jnp.dot(x_ref[...], Wr_ref[...].T, preferred_element_type=jnp.float32)
    )
    o_ref[...] = jnp.maximum(o, 0.0)


def _layer2_kernel(x_ref, h_ref, agg_ref, Wl_ref, bl_ref, Wr_ref, g_ref, b_ref, o_ref):
    agg = agg_ref[...]
    agg = jnp.where(jnp.isfinite(agg), agg, 0.0)
    h2 = (
        jnp.dot(agg, Wl_ref[...].T, preferred_element_type=jnp.float32)
        + bl_ref[...]
        + jnp.dot(h_ref[...], Wr_ref[...].T, preferred_element_type=jnp.float32)
        + x_ref[...]
    )
    mean = jnp.mean(h2, axis=-1, keepdims=True)
    var = jnp.mean((h2 - mean) ** 2, axis=-1, keepdims=True)
    o_ref[...] = (h2 - mean) * jax.lax.rsqrt(var + 1e-5) * g_ref[...] + b_ref[...]


def _row_spec():
    return pl.BlockSpec((ROW_BLK, D_), lambda i: (i, 0))


def _full_spec(shape):
    return pl.BlockSpec(shape, lambda i: tuple(0 for _ in shape))


def _dense1(x, agg, Wl, bl, Wr):
    return pl.pallas_call(
        _layer1_kernel,
        grid=(N_NODES_ // ROW_BLK,),
        in_specs=[
            _row_spec(),
            _row_spec(),
            _full_spec((D_, D_)),
            _full_spec((1, D_)),
            _full_spec((D_, D_)),
        ],
        out_specs=_row_spec(),
        out_shape=jax.ShapeDtypeStruct((N_NODES_, D_), jnp.float32),
    )(x, agg, Wl, bl.reshape(1, D_), Wr)


def _dense2(x, h, agg, Wl, bl, Wr, gamma, beta):
    return pl.pallas_call(
        _layer2_kernel,
        grid=(N_NODES_ // ROW_BLK,),
        in_specs=[
            _row_spec(),
            _row_spec(),
            _row_spec(),
            _full_spec((D_, D_)),
            _full_spec((1, D_)),
            _full_spec((D_, D_)),
            _full_spec((1, D_)),
            _full_spec((1, D_)),
        ],
        out_specs=_row_spec(),
        out_shape=jax.ShapeDtypeStruct((N_NODES_, D_), jnp.float32),
    )(x, h, agg, Wl, bl.reshape(1, D_), Wr, gamma.reshape(1, D_), beta.reshape(1, D_))


def kernel(x, edge_index, W1l, b1l, W1r, W2l, b2l, W2r, gamma, beta):
    src = edge_index[0]
    dst = edge_index[1]
    bkt, cnt = _partition(src, dst)
    agg1 = _aggregate(x, bkt, cnt).reshape(AGG_PAD, D_)[:N_NODES_]
    h = _dense1(x, agg1, W1l, b1l, W1r)
    agg2 = _aggregate(h, bkt, cnt).reshape(AGG_PAD, D_)[:N_NODES_]
    return _dense2(x, h, agg2, W2l, b2l, W2r, gamma, beta)


# speedup vs baseline: 14.5740x; 14.5740x over previous
import dataclasses
import jax
import jax.numpy as jnp
from jax import lax
from jax.experimental import pallas as pl
from jax.experimental.pallas import tpu as pltpu
from jax.experimental.pallas import tpu_sc as plsc

N_NODES_ = 10000
D_ = 128
N_EDGES_ = 320000
ROW_BLK = 2000

NW = 32
BN = 320
EPW = N_EDGES_ // NW
BIG = N_EDGES_ + 768
DUMP = N_EDGES_ + 512
ACC_ROWS = BN + 1
AGG_PAD = NW * BN
NSTART = 33 * 32

_mesh = plsc.VectorSubcoreMesh(core_axis_name="c", subcore_axis_name="s")
_cp = pltpu.CompilerParams()
if "needs_layout_passes" in pltpu.CompilerParams.__dataclass_fields__:
    _cp = dataclasses.replace(_cp, needs_layout_passes=False)


def _partition(src, dst):
    @pl.kernel(
        out_type=(
            jax.ShapeDtypeStruct((BIG,), jnp.int32),
            jax.ShapeDtypeStruct((NSTART,), jnp.int32),
        ),
        mesh=_mesh,
        compiler_params=_cp,
        scratch_types=[
            pltpu.VMEM((EPW,), jnp.int32),
            pltpu.VMEM((EPW,), jnp.int32),
            pltpu.VMEM((N_NODES_ + 16,), jnp.int32),
            pltpu.VMEM((80, 128), jnp.int32),
            pltpu.VMEM((80, 128), jnp.int32),
            pltpu.VMEM((1, 48), jnp.int32),
            pltpu.VMEM((1, 48), jnp.int32),
            pltpu.SemaphoreType.DMA,
        ],
    )
    def k(src_hbm, dst_hbm, bkt_hbm, st_hbm, sv, dv, hist, posb, pakb, bvals, bpos, sem):
        wid = lax.axis_index("s") * 2 + lax.axis_index("c")
        iota = lax.iota(jnp.int32, 16)
        zeros16 = jnp.zeros((16,), jnp.int32)
        base_e = wid * EPW
        pltpu.sync_copy(src_hbm.at[pl.ds(base_e, EPW)], sv)
        pltpu.sync_copy(dst_hbm.at[pl.ds(base_e, EPW)], dv)

        def zbody(i, _):
            hist[pl.ds(16 * i, 16)] = zeros16
            return 0

        lax.fori_loop(0, (N_NODES_ + 16) // 16, zbody, 0)

        def hbody(i, _):
            d16 = dv[pl.ds(16 * i, 16)]
            cnt16, last16 = plsc.scan_count(d16)
            plsc.addupdate_scatter(hist, [d16], cnt16, mask=last16)
            return 0

        lax.fori_loop(0, EPW // 16, hbody, 0)

        def sbody(i, c):
            h = hist[pl.ds(16 * i, 16)]
            cs = plsc.cumsum(h)
            hist[pl.ds(16 * i, 16)] = cs - h + c
            return c + jnp.sum(h)

        lax.fori_loop(0, N_NODES_ // 16, sbody, jnp.int32(0))

        bvals[0, pl.ds(0, 16)] = plsc.load_gather(hist, [iota * BN])
        bvals[0, pl.ds(16, 16)] = plsc.load_gather(hist, [(iota + 16) * BN])
        bvals[0, pl.ds(32, 16)] = zeros16 + EPW
        bpos[0, pl.ds(0, 16)] = iota * 32 + wid
        bpos[0, pl.ds(16, 16)] = (iota + 16) * 32 + wid
        bpos[0, pl.ds(32, 16)] = zeros16 + 32 * 32 + wid
        cp0 = pltpu.async_copy(bvals.at[0], st_hbm.at[bpos.at[0]], sem)

        for j in (78, 79):
            for c in range(8):
                posb[j, pl.ds(16 * c, 16)] = zeros16 + DUMP

        def pbody(i, _):
            s16 = sv[pl.ds(16 * i, 16)]
            d16 = dv[pl.ds(16 * i, 16)]
            b16 = (d16 * 6554) >> 21
            dl16 = d16 - b16 * BN
            cnt16, last16 = plsc.scan_count(d16)
            start16 = plsc.load_gather(hist, [d16])
            pos16 = base_e + start16 + cnt16 - 1
            plsc.addupdate_scatter(hist, [d16], cnt16, mask=last16)
            pk16 = (s16 << 9) | dl16
            r = i >> 3
            col = (i & 7) * 16
            posb[r, pl.ds(col, 16)] = pos16
            pakb[r, pl.ds(col, 16)] = pk16
            return 0

        lax.fori_loop(0, EPW // 16, pbody, 0)
        cps = [
            pltpu.async_copy(pakb.at[j], bkt_hbm.at[posb.at[j]], sem)
            for j in range(80)
        ]
        cp0.wait()
        for cp in cps:
            cp.wait()

    return k(src, dst)


def _aggregate(feats, bkt, cnt):
    @pl.kernel(
        out_type=jax.ShapeDtypeStruct((AGG_PAD * D_,), jnp.float32),
        mesh=_mesh,
        compiler_params=_cp,
        scratch_types=[
            pltpu.VMEM((ACC_ROWS * D_,), jnp.float32),
            pltpu.VMEM((512, D_), jnp.float32),
            pltpu.VMEM((512,), jnp.int32),
            pltpu.VMEM((4, 128), jnp.int32),
            pltpu.VMEM((528,), jnp.int32),
            pltpu.VMEM((1088,), jnp.int32),
            pltpu.SemaphoreType.DMA,
        ],
    )
    def k(f_hbm, bkt_hbm, st_hbm, agg_hbm, acc, rows, pkv, idxb, dlv, cntv, sem):
        b = lax.axis_index("s") * 2 + lax.axis_index("c")
        iota = lax.iota(jnp.int32, 16)
        pltpu.sync_copy(st_hbm, cntv.at[pl.ds(0, NSTART)])
        neg = jnp.full((16,), -jnp.inf, jnp.float32)
        dlv[pl.ds(512, 16)] = jnp.zeros((16,), jnp.int32) + BN

        def initbody(i, _):
            acc[pl.ds(16 * i, 16)] = neg
            return 0

        lax.fori_loop(0, ACC_ROWS * D_ // 16, initbody, 0)

        def wbody(w, _):
            s0 = cntv[pl.ds(b * 32 + w, 16)][0]
            s1 = cntv[pl.ds((b + 1) * 32 + w, 16)][0]
            nspan = s1 - s0
            lo0 = s0 & 7
            start_al = s0 - lo0
            total = s1 - start_al
            nch = jnp.where(nspan > 0, (total + 511) >> 9, 0)

            def cbody(i, _):
                moff = pl.multiple_of(w * EPW + start_al + 512 * i, 8)
                pltpu.sync_copy(bkt_hbm.at[pl.ds(moff, 512)], pkv)
                lo = jnp.where(i == 0, lo0, 0)
                hi = jnp.minimum(512, total - 512 * i)
                for vv in range(32):
                    pk = pkv[pl.ds(16 * vv, 16)]
                    lpos = iota + 16 * vv
                    act = (lpos >= lo) & (lpos < hi)
                    pad = (lpos * 577 + 131 * i) & 8191
                    idxb[vv >> 3, pl.ds((vv & 7) * 16, 16)] = jnp.where(
                        act, pk >> 9, pad
                    )
                    dlv[pl.ds(16 * vv, 16)] = jnp.where(act, pk & 511, BN)
                ng = (hi + 127) >> 7

                def gbody(g, _):
                    pltpu.async_copy(
                        f_hbm.at[idxb.at[g]], rows.at[pl.ds(128 * g, 128)], sem
                    )
                    return 0

                lax.fori_loop(0, ng, gbody, 0)

                def dbody(g, _):
                    pltpu.make_async_copy(
                        f_hbm.at[idxb.at[g]], rows.at[pl.ds(128 * g, 128)], sem
                    ).wait()
                    return 0

                lax.fori_loop(0, ng, dbody, 0)

                d0 = dlv[pl.ds(lo, 16)][0]
                regs0 = tuple(
                    acc[pl.ds(d0 * D_ + 16 * c, 16)] for c in range(8)
                )

                def ebody(e, regs):
                    dv2 = dlv[pl.ds(e, 16)]
                    d = dv2[0]
                    dn = dv2[1]
                    mx = tuple(
                        jnp.maximum(regs[c], rows[e, pl.ds(16 * c, 16)])
                        for c in range(8)
                    )
                    flush = dn != d

                    @pl.when(flush)
                    def _():
                        ab = d * D_
                        for c in range(8):
                            acc[pl.ds(ab + 16 * c, 16)] = mx[c]

                    nregs = lax.cond(
                        flush,
                        lambda: tuple(
                            acc[pl.ds(dn * D_ + 16 * c, 16)] for c in range(8)
                        ),
                        lambda: mx,
                    )
                    return nregs

                lax.fori_loop(lo, hi, ebody, regs0)
                return 0

            lax.fori_loop(0, nch, cbody, 0)
            return 0

        lax.fori_loop(0, NW, wbody, 0)
        pltpu.sync_copy(
            acc.at[pl.ds(0, BN * D_)], agg_hbm.at[pl.ds(b * BN * D_, BN * D_)]
        )

    return k(feats, bkt, cnt)


def _layer1_kernel(x_ref, agg_ref, Wl_ref, bl_ref, Wr_ref, o_ref):
    agg = agg_ref[...]
    agg = jnp.where(jnp.isfinite(agg), agg, 0.0)
    o = (
        jnp.dot(agg, Wl_ref[...].T, preferred_element_type=jnp.float32)
        + bl_ref[...]
        + jnp.dot(x_ref[...], Wr_ref[...].T, preferred_element_type=jnp.float32)
    )
    o_ref[...] = jnp.maximum(o, 0.0)


def _layer2_kernel(x_ref, h_ref, agg_ref, Wl_ref, bl_ref, Wr_ref, g_ref, b_ref, o_ref):
    agg = agg_ref[...]
    agg = jnp.where(jnp.isfinite(agg), agg, 0.0)
    h2 = (
        jnp.dot(agg, Wl_ref[...].T, preferred_element_type=jnp.float32)
        + bl_ref[...]
        + jnp.dot(h_ref[...], Wr_ref[...].T, preferred_element_type=jnp.float32)
        + x_ref[...]
    )
    mean = jnp.mean(h2, axis=-1, keepdims=True)
    var = jnp.mean((h2 - mean) ** 2, axis=-1, keepdims=True)
    o_ref[...] = (h2 - mean) * jax.lax.rsqrt(var + 1e-5) * g_ref[...] + b_ref[...]


def _row_spec():
    return pl.BlockSpec((ROW_BLK, D_), lambda i: (i, 0))


def _full_spec(shape):
    return pl.BlockSpec(shape, lambda i: tuple(0 for _ in shape))


def _dense1(x, agg, Wl, bl, Wr):
    return pl.pallas_call(
        _layer1_kernel,
        grid=(N_NODES_ // ROW_BLK,),
        in_specs=[
            _row_spec(),
            _row_spec(),
            _full_spec((D_, D_)),
            _full_spec((1, D_)),
            _full_spec((D_, D_)),
        ],
        out_specs=_row_spec(),
        out_shape=jax.ShapeDtypeStruct((N_NODES_, D_), jnp.float32),
    )(x, agg, Wl, bl.reshape(1, D_), Wr)


def _dense2(x, h, agg, Wl, bl, Wr, gamma, beta):
    return pl.pallas_call(
        _layer2_kernel,
        grid=(N_NODES_ // ROW_BLK,),
        in_specs=[
            _row_spec(),
            _row_spec(),
            _row_spec(),
            _full_spec((D_, D_)),
            _full_spec((1, D_)),
            _full_spec((D_, D_)),
            _full_spec((1, D_)),
            _full_spec((1, D_)),
        ],
        out_specs=_row_spec(),
        out_shape=jax.ShapeDtypeStruct((N_NODES_, D_), jnp.float32),
    )(x, h, agg, Wl, bl.reshape(1, D_), Wr, gamma.reshape(1, D_), beta.reshape(1, D_))


def kernel(x, edge_index, W1l, b1l, W1r, W2l, b2l, W2r, gamma, beta):
    src = edge_index[0]
    dst = edge_index[1]
    bkt, cnt = _partition(src, dst)
    agg1 = _aggregate(x, bkt, cnt).reshape(AGG_PAD, D_)[:N_NODES_]
    h = _dense1(x, agg1, W1l, b1l, W1r)
    agg2 = _aggregate(h, bkt, cnt).reshape(AGG_PAD, D_)[:N_NODES_]
    return _dense2(x, h, agg2, W2l, b2l, W2r, gamma, beta)


# speedup vs baseline: 39.7873x; 2.7300x over previous
import dataclasses
import jax
import jax.numpy as jnp
from jax import lax
from jax.experimental import pallas as pl
from jax.experimental.pallas import tpu as pltpu
from jax.experimental.pallas import tpu_sc as plsc

N_NODES_ = 10000
D_ = 128
N_EDGES_ = 320000
ROW_BLK = 2000

NW = 32
BN = 320
EPW = N_EDGES_ // NW
BIG = N_EDGES_ + 768
DUMP = N_EDGES_ + 512
ACC_ROWS = BN + 1
AGG_PAD = NW * BN
NSTART = 33 * 32

_mesh = plsc.VectorSubcoreMesh(core_axis_name="c", subcore_axis_name="s")
_cp = pltpu.CompilerParams()
if "needs_layout_passes" in pltpu.CompilerParams.__dataclass_fields__:
    _cp = dataclasses.replace(_cp, needs_layout_passes=False)


def _partition(src, dst):
    @pl.kernel(
        out_type=(
            jax.ShapeDtypeStruct((BIG,), jnp.int32),
            jax.ShapeDtypeStruct((NSTART,), jnp.int32),
        ),
        mesh=_mesh,
        compiler_params=_cp,
        scratch_types=[
            pltpu.VMEM((EPW,), jnp.int32),
            pltpu.VMEM((EPW,), jnp.int32),
            pltpu.VMEM((N_NODES_ + 16,), jnp.int32),
            pltpu.VMEM((EPW + 16,), jnp.int32),
            pltpu.VMEM((1, 48), jnp.int32),
            pltpu.VMEM((1, 48), jnp.int32),
            pltpu.SemaphoreType.DMA,
        ],
    )
    def k(src_hbm, dst_hbm, bkt_hbm, st_hbm, sv, dv, hist, srt, bvals, bpos, sem):
        wid = lax.axis_index("s") * 2 + lax.axis_index("c")
        iota = lax.iota(jnp.int32, 16)
        zeros16 = jnp.zeros((16,), jnp.int32)
        base_e = wid * EPW
        pltpu.sync_copy(src_hbm.at[pl.ds(base_e, EPW)], sv)
        pltpu.sync_copy(dst_hbm.at[pl.ds(base_e, EPW)], dv)

        def zbody(i, _):
            hist[pl.ds(16 * i, 16)] = zeros16
            return 0

        lax.fori_loop(0, (N_NODES_ + 16) // 16, zbody, 0)

        def hbody(i, _):
            d16 = dv[pl.ds(16 * i, 16)]
            cnt16, last16 = plsc.scan_count(d16)
            plsc.addupdate_scatter(hist, [d16], cnt16, mask=last16)
            return 0

        lax.fori_loop(0, EPW // 16, hbody, 0)

        def sbody(i, c):
            h = hist[pl.ds(16 * i, 16)]
            cs = plsc.cumsum(h)
            hist[pl.ds(16 * i, 16)] = cs - h + c
            return c + jnp.sum(h)

        lax.fori_loop(0, N_NODES_ // 16, sbody, jnp.int32(0))

        bvals[0, pl.ds(0, 16)] = plsc.load_gather(hist, [iota * BN])
        bvals[0, pl.ds(16, 16)] = plsc.load_gather(hist, [(iota + 16) * BN])
        bvals[0, pl.ds(32, 16)] = zeros16 + EPW
        bpos[0, pl.ds(0, 16)] = iota * 32 + wid
        bpos[0, pl.ds(16, 16)] = (iota + 16) * 32 + wid
        bpos[0, pl.ds(32, 16)] = zeros16 + 32 * 32 + wid
        cp0 = pltpu.async_copy(bvals.at[0], st_hbm.at[bpos.at[0]], sem)

        def pbody(i, _):
            s16 = sv[pl.ds(16 * i, 16)]
            d16 = dv[pl.ds(16 * i, 16)]
            b16 = (d16 * 6554) >> 21
            dl16 = d16 - b16 * BN
            cnt16, last16 = plsc.scan_count(d16)
            start16 = plsc.load_gather(hist, [d16])
            pos16 = start16 + cnt16 - 1
            plsc.addupdate_scatter(hist, [d16], cnt16, mask=last16)
            pk16 = (s16 << 9) | dl16
            plsc.store_scatter(srt, [pos16], pk16)
            return 0

        lax.fori_loop(0, EPW // 16, pbody, 0)
        pltpu.sync_copy(srt.at[pl.ds(0, EPW)], bkt_hbm.at[pl.ds(base_e, EPW)])
        cp0.wait()

    return k(src, dst)


def _aggregate(feats, bkt, cnt):
    @pl.kernel(
        out_type=jax.ShapeDtypeStruct((AGG_PAD * D_,), jnp.float32),
        mesh=_mesh,
        compiler_params=_cp,
        scratch_types=[
            pltpu.VMEM((ACC_ROWS * D_,), jnp.float32),
            pltpu.VMEM((512, D_), jnp.float32),
            pltpu.VMEM((512,), jnp.int32),
            pltpu.VMEM((4, 128), jnp.int32),
            pltpu.VMEM((528,), jnp.int32),
            pltpu.VMEM((1088,), jnp.int32),
            pltpu.SemaphoreType.DMA,
        ],
    )
    def k(f_hbm, bkt_hbm, st_hbm, agg_hbm, acc, rows, pkv, idxb, dlv, cntv, sem):
        b = lax.axis_index("s") * 2 + lax.axis_index("c")
        iota = lax.iota(jnp.int32, 16)
        pltpu.sync_copy(st_hbm, cntv.at[pl.ds(0, NSTART)])
        neg = jnp.full((16,), -jnp.inf, jnp.float32)
        dlv[pl.ds(512, 16)] = jnp.zeros((16,), jnp.int32) + BN

        def initbody(i, _):
            acc[pl.ds(16 * i, 16)] = neg
            return 0

        lax.fori_loop(0, ACC_ROWS * D_ // 16, initbody, 0)

        def wbody(w, _):
            s0 = cntv[pl.ds(b * 32 + w, 16)][0]
            s1 = cntv[pl.ds((b + 1) * 32 + w, 16)][0]
            nspan = s1 - s0
            lo0 = s0 & 7
            start_al = s0 - lo0
            total = s1 - start_al
            nch = jnp.where(nspan > 0, (total + 511) >> 9, 0)

            def cbody(i, _):
                moff = pl.multiple_of(w * EPW + start_al + 512 * i, 8)
                pltpu.sync_copy(bkt_hbm.at[pl.ds(moff, 512)], pkv)
                lo = jnp.where(i == 0, lo0, 0)
                hi = jnp.minimum(512, total - 512 * i)
                for vv in range(32):
                    pk = pkv[pl.ds(16 * vv, 16)]
                    lpos = iota + 16 * vv
                    act = (lpos >= lo) & (lpos < hi)
                    pad = (lpos * 577 + 131 * i) & 8191
                    idxb[vv >> 3, pl.ds((vv & 7) * 16, 16)] = jnp.where(
                        act, pk >> 9, pad
                    )
                    dlv[pl.ds(16 * vv, 16)] = jnp.where(act, pk & 511, BN)
                ng = (hi + 127) >> 7

                def gbody(g, _):
                    pltpu.async_copy(
                        f_hbm.at[idxb.at[g]], rows.at[pl.ds(128 * g, 128)], sem
                    )
                    return 0

                lax.fori_loop(0, ng, gbody, 0)

                def dbody(g, _):
                    pltpu.make_async_copy(
                        f_hbm.at[idxb.at[g]], rows.at[pl.ds(128 * g, 128)], sem
                    ).wait()
                    return 0

                lax.fori_loop(0, ng, dbody, 0)

                d0 = dlv[pl.ds(lo, 16)][0]
                regs0 = tuple(
                    acc[pl.ds(d0 * D_ + 16 * c, 16)] for c in range(8)
                )

                def ebody(e, regs):
                    dv2 = dlv[pl.ds(e, 16)]
                    d = dv2[0]
                    dn = dv2[1]
                    mx = tuple(
                        jnp.maximum(regs[c], rows[e, pl.ds(16 * c, 16)])
                        for c in range(8)
                    )
                    flush = dn != d

                    @pl.when(flush)
                    def _():
                        ab = d * D_
                        for c in range(8):
                            acc[pl.ds(ab + 16 * c, 16)] = mx[c]

                    nregs = lax.cond(
                        flush,
                        lambda: tuple(
                            acc[pl.ds(dn * D_ + 16 * c, 16)] for c in range(8)
                        ),
                        lambda: mx,
                    )
                    return nregs

                lax.fori_loop(lo, hi, ebody, regs0)
                return 0

            lax.fori_loop(0, nch, cbody, 0)
            return 0

        lax.fori_loop(0, NW, wbody, 0)
        pltpu.sync_copy(
            acc.at[pl.ds(0, BN * D_)], agg_hbm.at[pl.ds(b * BN * D_, BN * D_)]
        )

    return k(feats, bkt, cnt)


def _layer1_kernel(x_ref, agg_ref, Wl_ref, bl_ref, Wr_ref, o_ref):
    agg = agg_ref[...]
    agg = jnp.where(jnp.isfinite(agg), agg, 0.0)
    o = (
        jnp.dot(agg, Wl_ref[...].T, preferred_element_type=jnp.float32)
        + bl_ref[...]
        + jnp.dot(x_ref[...], Wr_ref[...].T, preferred_element_type=jnp.float32)
    )
    o_ref[...] = jnp.maximum(o, 0.0)


def _layer2_kernel(x_ref, h_ref, agg_ref, Wl_ref, bl_ref, Wr_ref, g_ref, b_ref, o_ref):
    agg = agg_ref[...]
    agg = jnp.where(jnp.isfinite(agg), agg, 0.0)
    h2 = (
        jnp.dot(agg, Wl_ref[...].T, preferred_element_type=jnp.float32)
        + bl_ref[...]
        + jnp.dot(h_ref[...], Wr_ref[...].T, preferred_element_type=jnp.float32)
        + x_ref[...]
    )
    mean = jnp.mean(h2, axis=-1, keepdims=True)
    var = jnp.mean((h2 - mean) ** 2, axis=-1, keepdims=True)
    o_ref[...] = (h2 - mean) * jax.lax.rsqrt(var + 1e-5) * g_ref[...] + b_ref[...]


def _row_spec():
    return pl.BlockSpec((ROW_BLK, D_), lambda i: (i, 0))


def _full_spec(shape):
    return pl.BlockSpec(shape, lambda i: tuple(0 for _ in shape))


def _dense1(x, agg, Wl, bl, Wr):
    return pl.pallas_call(
        _layer1_kernel,
        grid=(N_NODES_ // ROW_BLK,),
        in_specs=[
            _row_spec(),
            _row_spec(),
            _full_spec((D_, D_)),
            _full_spec((1, D_)),
            _full_spec((D_, D_)),
        ],
        out_specs=_row_spec(),
        out_shape=jax.ShapeDtypeStruct((N_NODES_, D_), jnp.float32),
    )(x, agg, Wl, bl.reshape(1, D_), Wr)


def _dense2(x, h, agg, Wl, bl, Wr, gamma, beta):
    return pl.pallas_call(
        _layer2_kernel,
        grid=(N_NODES_ // ROW_BLK,),
        in_specs=[
            _row_spec(),
            _row_spec(),
            _row_spec(),
            _full_spec((D_, D_)),
            _full_spec((1, D_)),
            _full_spec((D_, D_)),
            _full_spec((1, D_)),
            _full_spec((1, D_)),
        ],
        out_specs=_row_spec(),
        out_shape=jax.ShapeDtypeStruct((N_NODES_, D_), jnp.float32),
    )(x, h, agg, Wl, bl.reshape(1, D_), Wr, gamma.reshape(1, D_), beta.reshape(1, D_))


def kernel(x, edge_index, W1l, b1l, W1r, W2l, b2l, W2r, gamma, beta):
    src = edge_index[0]
    dst = edge_index[1]
    bkt, cnt = _partition(src, dst)
    agg1 = _aggregate(x, bkt, cnt).reshape(AGG_PAD, D_)[:N_NODES_]
    h = _dense1(x, agg1, W1l, b1l, W1r)
    agg2 = _aggregate(h, bkt, cnt).reshape(AGG_PAD, D_)[:N_NODES_]
    return _dense2(x, h, agg2, W2l, b2l, W2r, gamma, beta)


# speedup vs baseline: 44.0228x; 1.1065x over previous
import dataclasses
import jax
import jax.numpy as jnp
from jax import lax
from jax.experimental import pallas as pl
from jax.experimental.pallas import tpu as pltpu
from jax.experimental.pallas import tpu_sc as plsc

N_NODES_ = 10000
D_ = 128
N_EDGES_ = 320000
ROW_BLK = 2000

NW = 32
BN = 320
EPW = N_EDGES_ // NW
BIG = N_EDGES_ + 768
DUMP = N_EDGES_ + 512
ACC_ROWS = BN + 1
AGG_PAD = NW * BN
NSTART = 33 * 32

_mesh = plsc.VectorSubcoreMesh(core_axis_name="c", subcore_axis_name="s")
_cp = pltpu.CompilerParams()
if "needs_layout_passes" in pltpu.CompilerParams.__dataclass_fields__:
    _cp = dataclasses.replace(_cp, needs_layout_passes=False)


def _partition(src, dst):
    @pl.kernel(
        out_type=(
            jax.ShapeDtypeStruct((BIG,), jnp.int32),
            jax.ShapeDtypeStruct((NSTART,), jnp.int32),
        ),
        mesh=_mesh,
        compiler_params=_cp,
        scratch_types=[
            pltpu.VMEM((EPW,), jnp.int32),
            pltpu.VMEM((EPW,), jnp.int32),
            pltpu.VMEM((N_NODES_ + 16,), jnp.int32),
            pltpu.VMEM((EPW + 16,), jnp.int32),
            pltpu.VMEM((1, 48), jnp.int32),
            pltpu.VMEM((1, 48), jnp.int32),
            pltpu.SemaphoreType.DMA,
        ],
    )
    def k(src_hbm, dst_hbm, bkt_hbm, st_hbm, sv, dv, hist, srt, bvals, bpos, sem):
        wid = lax.axis_index("s") * 2 + lax.axis_index("c")
        iota = lax.iota(jnp.int32, 16)
        zeros16 = jnp.zeros((16,), jnp.int32)
        base_e = wid * EPW
        pltpu.sync_copy(src_hbm.at[pl.ds(base_e, EPW)], sv)
        pltpu.sync_copy(dst_hbm.at[pl.ds(base_e, EPW)], dv)

        def zbody(i, _):
            hist[pl.ds(16 * i, 16)] = zeros16
            return 0

        lax.fori_loop(0, (N_NODES_ + 16) // 16, zbody, 0)

        def hbody(i, _):
            d16 = dv[pl.ds(16 * i, 16)]
            cnt16, last16 = plsc.scan_count(d16)
            plsc.addupdate_scatter(hist, [d16], cnt16, mask=last16)
            return 0

        lax.fori_loop(0, EPW // 16, hbody, 0)

        def sbody(i, c):
            h = hist[pl.ds(16 * i, 16)]
            cs = plsc.cumsum(h)
            hist[pl.ds(16 * i, 16)] = cs - h + c
            return c + jnp.sum(h)

        lax.fori_loop(0, N_NODES_ // 16, sbody, jnp.int32(0))

        bvals[0, pl.ds(0, 16)] = plsc.load_gather(hist, [iota * BN])
        bvals[0, pl.ds(16, 16)] = plsc.load_gather(hist, [(iota + 16) * BN])
        bvals[0, pl.ds(32, 16)] = zeros16 + EPW
        bpos[0, pl.ds(0, 16)] = iota * 32 + wid
        bpos[0, pl.ds(16, 16)] = (iota + 16) * 32 + wid
        bpos[0, pl.ds(32, 16)] = zeros16 + 32 * 32 + wid
        cp0 = pltpu.async_copy(bvals.at[0], st_hbm.at[bpos.at[0]], sem)

        def pbody(i, _):
            s16 = sv[pl.ds(16 * i, 16)]
            d16 = dv[pl.ds(16 * i, 16)]
            b16 = (d16 * 6554) >> 21
            dl16 = d16 - b16 * BN
            cnt16, last16 = plsc.scan_count(d16)
            start16 = plsc.load_gather(hist, [d16])
            pos16 = start16 + cnt16 - 1
            plsc.addupdate_scatter(hist, [d16], cnt16, mask=last16)
            pk16 = (s16 << 9) | dl16
            plsc.store_scatter(srt, [pos16], pk16)
            return 0

        lax.fori_loop(0, EPW // 16, pbody, 0)
        pltpu.sync_copy(srt.at[pl.ds(0, EPW)], bkt_hbm.at[pl.ds(base_e, EPW)])
        cp0.wait()

    return k(src, dst)


def _aggregate(feats, bkt, cnt):
    @pl.kernel(
        out_type=jax.ShapeDtypeStruct((AGG_PAD * D_,), jnp.float32),
        mesh=_mesh,
        compiler_params=_cp,
        scratch_types=[
            pltpu.VMEM((ACC_ROWS * D_,), jnp.float32),
            pltpu.VMEM((512, D_), jnp.float32),
            pltpu.VMEM((512,), jnp.int32),
            pltpu.VMEM((4, 128), jnp.int32),
            pltpu.VMEM((528,), jnp.int32),
            pltpu.VMEM((1088,), jnp.int32),
            pltpu.SemaphoreType.DMA,
        ],
    )
    def k(f_hbm, bkt_hbm, st_hbm, agg_hbm, acc, rows, pkv, idxb, dlv, cntv, sem):
        b = lax.axis_index("s") * 2 + lax.axis_index("c")
        iota = lax.iota(jnp.int32, 16)
        pltpu.sync_copy(st_hbm, cntv.at[pl.ds(0, NSTART)])
        neg = jnp.full((16,), -jnp.inf, jnp.float32)
        dlv[pl.ds(512, 16)] = jnp.zeros((16,), jnp.int32) + BN

        def initbody(i, _):
            acc[pl.ds(16 * i, 16)] = neg
            return 0

        lax.fori_loop(0, ACC_ROWS * D_ // 16, initbody, 0)

        def wbody(w, _):
            s0 = cntv[pl.ds(b * 32 + w, 16)][0]
            s1 = cntv[pl.ds((b + 1) * 32 + w, 16)][0]
            nspan = s1 - s0
            lo0 = s0 & 7
            start_al = s0 - lo0
            total = s1 - start_al
            nch = jnp.where(nspan > 0, (total + 511) >> 9, 0)

            def cbody(i, _):
                moff = pl.multiple_of(w * EPW + start_al + 512 * i, 8)
                pltpu.sync_copy(bkt_hbm.at[pl.ds(moff, 512)], pkv)
                lo = jnp.where(i == 0, lo0, 0)
                hi = jnp.minimum(512, total - 512 * i)
                for vv in range(32):
                    pk = pkv[pl.ds(16 * vv, 16)]
                    lpos = iota + 16 * vv
                    act = (lpos >= lo) & (lpos < hi)
                    pad = (lpos * 577 + 131 * i) & 8191
                    idxb[vv >> 3, pl.ds((vv & 7) * 16, 16)] = jnp.where(
                        act, pk >> 9, pad
                    )
                    dlv[pl.ds(16 * vv, 16)] = jnp.where(act, pk & 511, BN)
                ng = (hi + 127) >> 7

                def gbody(g, _):
                    pltpu.async_copy(
                        f_hbm.at[idxb.at[g]], rows.at[pl.ds(128 * g, 128)], sem
                    )
                    return 0

                lax.fori_loop(0, ng, gbody, 0)

                d0 = dlv[pl.ds(lo, 16)][0]
                regs0 = tuple(
                    acc[pl.ds(d0 * D_ + 16 * c, 16)] for c in range(8)
                )

                def ebody(e, regs):
                    dv2 = dlv[pl.ds(e, 16)]
                    d = dv2[0]
                    dn = dv2[1]
                    mx = tuple(
                        jnp.maximum(regs[c], rows[e, pl.ds(16 * c, 16)])
                        for c in range(8)
                    )
                    flush = dn != d

                    @pl.when(flush)
                    def _():
                        ab = d * D_
                        for c in range(8):
                            acc[pl.ds(ab + 16 * c, 16)] = mx[c]

                    nregs = lax.cond(
                        flush,
                        lambda: tuple(
                            acc[pl.ds(dn * D_ + 16 * c, 16)] for c in range(8)
                        ),
                        lambda: mx,
                    )
                    return nregs

                def gproc(g, regs):
                    pltpu.make_async_copy(
                        f_hbm.at[idxb.at[g]], rows.at[pl.ds(128 * g, 128)], sem
                    ).wait()
                    e0 = jnp.maximum(lo, 128 * g)
                    e1 = jnp.minimum(hi, 128 * (g + 1))
                    return lax.fori_loop(e0, e1, ebody, regs)

                lax.fori_loop(0, ng, gproc, regs0)
                return 0

            lax.fori_loop(0, nch, cbody, 0)
            return 0

        lax.fori_loop(0, NW, wbody, 0)
        pltpu.sync_copy(
            acc.at[pl.ds(0, BN * D_)], agg_hbm.at[pl.ds(b * BN * D_, BN * D_)]
        )

    return k(feats, bkt, cnt)


def _layer1_kernel(x_ref, agg_ref, Wl_ref, bl_ref, Wr_ref, o_ref):
    agg = agg_ref[...]
    agg = jnp.where(jnp.isfinite(agg), agg, 0.0)
    o = (
        jnp.dot(agg, Wl_ref[...].T, preferred_element_type=jnp.float32)
        + bl_ref[...]
        + jnp.dot(x_ref[...], Wr_ref[...].T, preferred_element_type=jnp.float32)
    )
    o_ref[...] = jnp.maximum(o, 0.0)


def _layer2_kernel(x_ref, h_ref, agg_ref, Wl_ref, bl_ref, Wr_ref, g_ref, b_ref, o_ref):
    agg = agg_ref[...]
    agg = jnp.where(jnp.isfinite(agg), agg, 0.0)
    h2 = (
        jnp.dot(agg, Wl_ref[...].T, preferred_element_type=jnp.float32)
        + bl_ref[...]
        + jnp.dot(h_ref[...], Wr_ref[...].T, preferred_element_type=jnp.float32)
        + x_ref[...]
    )
    mean = jnp.mean(h2, axis=-1, keepdims=True)
    var = jnp.mean((h2 - mean) ** 2, axis=-1, keepdims=True)
    o_ref[...] = (h2 - mean) * jax.lax.rsqrt(var + 1e-5) * g_ref[...] + b_ref[...]


def _row_spec():
    return pl.BlockSpec((ROW_BLK, D_), lambda i: (i, 0))


def _full_spec(shape):
    return pl.BlockSpec(shape, lambda i: tuple(0 for _ in shape))


def _dense1(x, agg, Wl, bl, Wr):
    return pl.pallas_call(
        _layer1_kernel,
        grid=(N_NODES_ // ROW_BLK,),
        in_specs=[
            _row_spec(),
            _row_spec(),
            _full_spec((D_, D_)),
            _full_spec((1, D_)),
            _full_spec((D_, D_)),
        ],
        out_specs=_row_spec(),
        out_shape=jax.ShapeDtypeStruct((N_NODES_, D_), jnp.float32),
    )(x, agg, Wl, bl.reshape(1, D_), Wr)


def _dense2(x, h, agg, Wl, bl, Wr, gamma, beta):
    return pl.pallas_call(
        _layer2_kernel,
        grid=(N_NODES_ // ROW_BLK,),
        in_specs=[
            _row_spec(),
            _row_spec(),
            _row_spec(),
            _full_spec((D_, D_)),
            _full_spec((1, D_)),
            _full_spec((D_, D_)),
            _full_spec((1, D_)),
            _full_spec((1, D_)),
        ],
        out_specs=_row_spec(),
        out_shape=jax.ShapeDtypeStruct((N_NODES_, D_), jnp.float32),
    )(x, h, agg, Wl, bl.reshape(1, D_), Wr, gamma.reshape(1, D_), beta.reshape(1, D_))


def kernel(x, edge_index, W1l, b1l, W1r, W2l, b2l, W2r, gamma, beta):
    src = edge_index[0]
    dst = edge_index[1]
    bkt, cnt = _partition(src, dst)
    agg1 = _aggregate(x, bkt, cnt).reshape(AGG_PAD, D_)[:N_NODES_]
    h = _dense1(x, agg1, W1l, b1l, W1r)
    agg2 = _aggregate(h, bkt, cnt).reshape(AGG_PAD, D_)[:N_NODES_]
    return _dense2(x, h, agg2, W2l, b2l, W2r, gamma, beta)
